# baseline (speedup 1.0000x reference)
import functools

import jax
import jax.numpy as jnp
from jax import lax
from jax.experimental import pallas as pl
from jax.experimental.pallas import tpu as pltpu
from jax.experimental.pallas import tpu_sc as plsc

NC = 2
NS = 16
NW = NC * NS
LANES = 16


def _make_pool_kernel(B, L, D, V):
    assert B % NW == 0
    bpw = B // NW
    assert L % 2 == 0
    half = L // 2
    assert half <= 128
    nvec = D // LANES

    mesh = plsc.VectorSubcoreMesh(core_axis_name="c", subcore_axis_name="s")

    @functools.partial(
        pl.kernel,
        out_type=jax.ShapeDtypeStruct((B, D), jnp.float32),
        mesh=mesh,
        scratch_types=[
            pltpu.VMEM((bpw, 2, half), jnp.int32),
            pltpu.VMEM((2, 2, half, D), jnp.float32),
            pltpu.VMEM((bpw, D), jnp.float32),
            pltpu.SemaphoreType.DMA,
            pltpu.SemaphoreType.DMA,
        ],
        compiler_params=pltpu.CompilerParams(use_tc_tiling_on_sc=False),
    )
    def pool(idx_hbm, table_hbm, out_hbm, idx_v, rows_v, acc_v, sem0, sem1):
        wid = lax.axis_index("s") * NC + lax.axis_index("c")
        base = wid * bpw
        pltpu.sync_copy(idx_hbm.at[pl.ds(base, bpw)], idx_v)

        sems = (sem0, sem1)

        def issue(b, buf):
            for c in range(2):
                pltpu.async_copy(
                    table_hbm.at[idx_v.at[b, c]], rows_v.at[buf, c], sems[buf]
                )

        def wait(b, buf):
            for c in range(2):
                pltpu.make_async_copy(
                    table_hbm.at[idx_v.at[b, c]], rows_v.at[buf, c], sems[buf]
                ).wait()

        def reduce_into(buf, b):
            def body(j, accs):
                out = []
                for k in range(nvec):
                    v = accs[k]
                    for c in range(2):
                        v = v + rows_v[buf, c, j, pl.ds(LANES * k, LANES)]
                    out.append(v)
                return tuple(out)

            accs = lax.fori_loop(
                0, half, body,
                tuple(jnp.zeros((LANES,), jnp.float32) for _ in range(nvec)),
            )
            for k in range(nvec):
                acc_v[b, pl.ds(LANES * k, LANES)] = accs[k]

        issue(0, 0)

        def pair(g, carry):
            b0 = 2 * g

            issue(b0 + 1, 1)
            wait(b0, 0)
            reduce_into(0, b0)

            @pl.when(b0 + 2 < bpw)
            def _():
                issue(b0 + 2, 0)

            wait(b0 + 1, 1)
            reduce_into(1, b0 + 1)
            return carry

        lax.fori_loop(0, bpw // 2, pair, 0)

        pltpu.sync_copy(acc_v, out_hbm.at[pl.ds(base, bpw)])

    return pool


def _mlp_block(x_ref, w1_ref, b1_ref, w2_ref, b2_ref, w3_ref, b3_ref, o_ref):
    h = jnp.dot(x_ref[...], w1_ref[...], preferred_element_type=jnp.float32)
    h = jnp.maximum(h + b1_ref[...], 0.0)
    h = jnp.dot(h, w2_ref[...], preferred_element_type=jnp.float32)
    h = jnp.maximum(h + b2_ref[...], 0.0)
    o_ref[...] = (
        jnp.dot(h, w3_ref[...], preferred_element_type=jnp.float32) + b3_ref[...]
    )


def _mlp(pooled, W1, b1, W2, b2, W3, b3):
    B, D = pooled.shape
    H = W1.shape[1]
    O = W3.shape[1]
    BM = 512
    grid = (B // BM,)
    return pl.pallas_call(
        _mlp_block,
        grid=grid,
        in_specs=[
            pl.BlockSpec((BM, D), lambda i: (i, 0)),
            pl.BlockSpec((D, H), lambda i: (0, 0)),
            pl.BlockSpec((1, H), lambda i: (0, 0)),
            pl.BlockSpec((H, H), lambda i: (0, 0)),
            pl.BlockSpec((1, H), lambda i: (0, 0)),
            pl.BlockSpec((H, O), lambda i: (0, 0)),
            pl.BlockSpec((1, O), lambda i: (0, 0)),
        ],
        out_specs=pl.BlockSpec((BM, O), lambda i: (i, 0)),
        out_shape=jax.ShapeDtypeStruct((B, O), jnp.float32),
    )(pooled, W1, b1.reshape(1, H), W2, b2.reshape(1, H), W3, b3.reshape(1, O))


@jax.jit
def kernel(inputs, table, W1, b1, W2, b2, W3, b3):
    B, L = inputs.shape
    V, D = table.shape
    idx = inputs.astype(jnp.int32).reshape(B, 2, L // 2)
    pooled = _make_pool_kernel(B, L, D, V)(idx, table)
    return _mlp(pooled, W1, b1, W2, b2, W3, b3)


# speedup vs baseline: 1.2115x; 1.2115x over previous
import functools

import jax
import jax.numpy as jnp
from jax import lax
from jax.experimental import pallas as pl
from jax.experimental.pallas import tpu as pltpu
from jax.experimental.pallas import tpu_sc as plsc

NC = 2
NS = 16
NW = NC * NS
LANES = 16


_CB = 1024


def _transpose_block(x_ref, o_ref):
    x = x_ref[...]
    o_ref[:, 0:64] = x[:, 0:_CB].T
    o_ref[:, 64:128] = x[:, _CB : 2 * _CB].T


def _relayout_table(tableT):
    D, V = tableT.shape
    nblk = pl.cdiv(V, 2 * _CB)
    return pl.pallas_call(
        _transpose_block,
        grid=(nblk,),
        in_specs=[pl.BlockSpec((D, 2 * _CB), lambda i: (0, i))],
        out_specs=pl.BlockSpec((_CB, 2 * D), lambda i: (i, 0)),
        out_shape=jax.ShapeDtypeStruct((nblk * _CB, 2 * D), jnp.float32),
    )(tableT)


def _make_pool_kernel(B, L, D, V):
    assert B % NW == 0
    bpw = B // NW
    assert L % 2 == 0
    half = L // 2
    assert half <= 128
    nvec = D // LANES

    mesh = plsc.VectorSubcoreMesh(core_axis_name="c", subcore_axis_name="s")

    @functools.partial(
        pl.kernel,
        out_type=jax.ShapeDtypeStruct((B, D), jnp.float32),
        mesh=mesh,
        scratch_types=[
            pltpu.VMEM((bpw, 2, half), jnp.int32),
            pltpu.VMEM((2, 2, half, D), jnp.float32),
            pltpu.VMEM((bpw, D), jnp.float32),
            pltpu.SemaphoreType.DMA,
            pltpu.SemaphoreType.DMA,
        ],
        compiler_params=pltpu.CompilerParams(use_tc_tiling_on_sc=False),
    )
    def pool(idx_hbm, table_hbm, out_hbm, idx_v, rows_v, acc_v, sem0, sem1):
        wid = lax.axis_index("s") * NC + lax.axis_index("c")
        base = wid * bpw
        pltpu.sync_copy(idx_hbm.at[pl.ds(base, bpw)], idx_v)

        sems = (sem0, sem1)

        def issue(b, buf):
            for c in range(2):
                pltpu.async_copy(
                    table_hbm.at[idx_v.at[b, c]], rows_v.at[buf, c], sems[buf]
                )

        def wait(b, buf):
            for c in range(2):
                pltpu.make_async_copy(
                    table_hbm.at[idx_v.at[b, c]], rows_v.at[buf, c], sems[buf]
                ).wait()

        def reduce_into(buf, b):
            def body(j, accs):
                out = []
                for k in range(nvec):
                    v = accs[k]
                    for c in range(2):
                        v = v + rows_v[buf, c, j, pl.ds(LANES * k, LANES)]
                    out.append(v)
                return tuple(out)

            accs = lax.fori_loop(
                0, half, body,
                tuple(jnp.zeros((LANES,), jnp.float32) for _ in range(nvec)),
            )
            for k in range(nvec):
                acc_v[b, pl.ds(LANES * k, LANES)] = accs[k]

        issue(0, 0)

        def pair(g, carry):
            b0 = 2 * g

            issue(b0 + 1, 1)
            wait(b0, 0)
            reduce_into(0, b0)

            @pl.when(b0 + 2 < bpw)
            def _():
                issue(b0 + 2, 0)

            wait(b0 + 1, 1)
            reduce_into(1, b0 + 1)
            return carry

        lax.fori_loop(0, bpw // 2, pair, 0)

        pltpu.sync_copy(acc_v, out_hbm.at[pl.ds(base, bpw)])

    return pool


def _mlp_block(x_ref, w1_ref, b1_ref, w2_ref, b2_ref, w3_ref, b3_ref, o_ref):
    h = jnp.dot(x_ref[...], w1_ref[...], preferred_element_type=jnp.float32)
    h = jnp.maximum(h + b1_ref[...], 0.0)
    h = jnp.dot(h, w2_ref[...], preferred_element_type=jnp.float32)
    h = jnp.maximum(h + b2_ref[...], 0.0)
    o_ref[...] = (
        jnp.dot(h, w3_ref[...], preferred_element_type=jnp.float32) + b3_ref[...]
    )


def _mlp(pooled, W1, b1, W2, b2, W3, b3):
    B, D = pooled.shape
    H = W1.shape[1]
    O = W3.shape[1]
    BM = 512
    grid = (B // BM,)
    return pl.pallas_call(
        _mlp_block,
        grid=grid,
        in_specs=[
            pl.BlockSpec((BM, D), lambda i: (i, 0)),
            pl.BlockSpec((D, H), lambda i: (0, 0)),
            pl.BlockSpec((1, H), lambda i: (0, 0)),
            pl.BlockSpec((H, H), lambda i: (0, 0)),
            pl.BlockSpec((1, H), lambda i: (0, 0)),
            pl.BlockSpec((H, O), lambda i: (0, 0)),
            pl.BlockSpec((1, O), lambda i: (0, 0)),
        ],
        out_specs=pl.BlockSpec((BM, O), lambda i: (i, 0)),
        out_shape=jax.ShapeDtypeStruct((B, O), jnp.float32),
    )(pooled, W1, b1.reshape(1, H), W2, b2.reshape(1, H), W3, b3.reshape(1, O))


@jax.jit
def kernel(inputs, table, W1, b1, W2, b2, W3, b3):
    B, L = inputs.shape
    V, D = table.shape
    packed = _relayout_table(table.T)
    Vlin = 2 * packed.shape[0]
    table_lin = jnp.reshape(packed, (Vlin, D))
    r = inputs.astype(jnp.int32)
    rem = r & (2 * _CB - 1)
    p = ((r >> 11) << 10) + (rem & (_CB - 1))
    v = (p << 1) + (rem >> 10)
    idx = v.reshape(B, 2, L // 2)
    pooled = _make_pool_kernel(B, L, D, Vlin)(idx, table_lin)
    return _mlp(pooled, W1, b1, W2, b2, W3, b3)


# speedup vs baseline: 1.6177x; 1.3352x over previous
import functools

import jax
import jax.numpy as jnp
from jax import lax
from jax.experimental import pallas as pl
from jax.experimental.pallas import tpu as pltpu
from jax.experimental.pallas import tpu_sc as plsc

NC = 2
NS = 16
NW = NC * NS
LANES = 16

_CB = 1024


def _pack_block(x_ref, o_ref):
    x = x_ref[...]
    u = lax.bitcast_convert_type(x, jnp.uint32)
    r = u + jnp.uint32(0x7FFF) + ((u >> 16) & jnp.uint32(1))
    w = (r[0:32, :] >> 16) | (r[32:64, :] & jnp.uint32(0xFFFF0000))
    for c in range(4):
        o_ref[:, 32 * c : 32 * (c + 1)] = w[:, c * _CB : (c + 1) * _CB].T


def _relayout_table(tableT):
    D, V = tableT.shape
    nblk = pl.cdiv(V, 4 * _CB)
    return pl.pallas_call(
        _pack_block,
        grid=(nblk,),
        in_specs=[pl.BlockSpec((D, 4 * _CB), lambda i: (0, i))],
        out_specs=pl.BlockSpec((_CB, 2 * D), lambda i: (i, 0)),
        out_shape=jax.ShapeDtypeStruct((nblk * _CB, 2 * D), jnp.uint32),
    )(tableT)


def _make_pool_kernel(B, L, D, V):
    assert B % NW == 0
    bpw = B // NW
    assert L % 2 == 0
    half = L // 2
    assert half <= 128
    nvec = D // (2 * LANES)

    mesh = plsc.VectorSubcoreMesh(core_axis_name="c", subcore_axis_name="s")

    @functools.partial(
        pl.kernel,
        out_type=jax.ShapeDtypeStruct((B, D), jnp.float32),
        mesh=mesh,
        scratch_types=[
            pltpu.VMEM((bpw, 2, half), jnp.int32),
            pltpu.VMEM((2, 2, half, D // 2), jnp.uint32),
            pltpu.VMEM((bpw, D), jnp.float32),
            pltpu.SemaphoreType.DMA,
            pltpu.SemaphoreType.DMA,
        ],
        compiler_params=pltpu.CompilerParams(use_tc_tiling_on_sc=False),
    )
    def pool(idx_hbm, table_hbm, out_hbm, idx_v, rows_v, acc_v, sem0, sem1):
        wid = lax.axis_index("s") * NC + lax.axis_index("c")
        base = wid * bpw
        pltpu.sync_copy(idx_hbm.at[pl.ds(base, bpw)], idx_v)

        sems = (sem0, sem1)

        def issue(b, buf):
            for c in range(2):
                pltpu.async_copy(
                    table_hbm.at[idx_v.at[b, c]], rows_v.at[buf, c], sems[buf]
                )

        def wait(b, buf):
            for c in range(2):
                pltpu.make_async_copy(
                    table_hbm.at[idx_v.at[b, c]], rows_v.at[buf, c], sems[buf]
                ).wait()

        def reduce_into(buf, b):
            hi_mask = jnp.full((LANES,), 0xFFFF0000, jnp.uint32)

            def body(j, accs):
                out = list(accs)
                for k in range(nvec):
                    for c in range(2):
                        u = rows_v[buf, c, j, pl.ds(LANES * k, LANES)]
                        lo = lax.bitcast_convert_type(u << 16, jnp.float32)
                        hi = lax.bitcast_convert_type(u & hi_mask, jnp.float32)
                        out[k] = out[k] + lo
                        out[nvec + k] = out[nvec + k] + hi
                return tuple(out)

            accs = lax.fori_loop(
                0, half, body,
                tuple(jnp.zeros((LANES,), jnp.float32) for _ in range(2 * nvec)),
            )
            for k in range(2 * nvec):
                acc_v[b, pl.ds(LANES * k, LANES)] = accs[k]

        issue(0, 0)

        def pair(g, carry):
            b0 = 2 * g

            issue(b0 + 1, 1)
            wait(b0, 0)
            reduce_into(0, b0)

            @pl.when(b0 + 2 < bpw)
            def _():
                issue(b0 + 2, 0)

            wait(b0 + 1, 1)
            reduce_into(1, b0 + 1)
            return carry

        lax.fori_loop(0, bpw // 2, pair, 0)

        pltpu.sync_copy(acc_v, out_hbm.at[pl.ds(base, bpw)])

    return pool


def _mlp_block(x_ref, w1_ref, b1_ref, w2_ref, b2_ref, w3_ref, b3_ref, o_ref):
    h = jnp.dot(x_ref[...], w1_ref[...], preferred_element_type=jnp.float32)
    h = jnp.maximum(h + b1_ref[...], 0.0)
    h = jnp.dot(h, w2_ref[...], preferred_element_type=jnp.float32)
    h = jnp.maximum(h + b2_ref[...], 0.0)
    o_ref[...] = (
        jnp.dot(h, w3_ref[...], preferred_element_type=jnp.float32) + b3_ref[...]
    )


def _mlp(pooled, W1, b1, W2, b2, W3, b3):
    B, D = pooled.shape
    H = W1.shape[1]
    O = W3.shape[1]
    BM = 512
    grid = (B // BM,)
    return pl.pallas_call(
        _mlp_block,
        grid=grid,
        in_specs=[
            pl.BlockSpec((BM, D), lambda i: (i, 0)),
            pl.BlockSpec((D, H), lambda i: (0, 0)),
            pl.BlockSpec((1, H), lambda i: (0, 0)),
            pl.BlockSpec((H, H), lambda i: (0, 0)),
            pl.BlockSpec((1, H), lambda i: (0, 0)),
            pl.BlockSpec((H, O), lambda i: (0, 0)),
            pl.BlockSpec((1, O), lambda i: (0, 0)),
        ],
        out_specs=pl.BlockSpec((BM, O), lambda i: (i, 0)),
        out_shape=jax.ShapeDtypeStruct((B, O), jnp.float32),
    )(pooled, W1, b1.reshape(1, H), W2, b2.reshape(1, H), W3, b3.reshape(1, O))


@jax.jit
def kernel(inputs, table, W1, b1, W2, b2, W3, b3):
    B, L = inputs.shape
    V, D = table.shape
    packed = _relayout_table(table.T)
    V4 = 4 * packed.shape[0]
    table_rows = jnp.reshape(packed, (V4, D // 2))
    r = inputs.astype(jnp.int32)
    v = ((r >> 12) << 12) + ((r & (4 * _CB - 1) & (_CB - 1)) << 2) + (
        (r >> 10) & 3
    )
    idx = v.reshape(B, 2, L // 2)
    pooled = _make_pool_kernel(B, L, D, V4)(idx, table_rows)
    return _mlp(pooled, W1, b1, W2, b2, W3, b3)


# speedup vs baseline: 1.8815x; 1.1631x over previous
import functools

import jax
import jax.numpy as jnp
from jax import lax
from jax.experimental import pallas as pl
from jax.experimental.pallas import tpu as pltpu
from jax.experimental.pallas import tpu_sc as plsc

NC = 2
NS = 16
NW = NC * NS
LANES = 16

_CB = 2048
_CBSH = 11


def _pack_block(x_ref, o_ref):
    x = x_ref[...]
    u = lax.bitcast_convert_type(x, jnp.uint32)
    r = u + jnp.uint32(0x7FFF) + ((u >> 16) & jnp.uint32(1))
    w = (r[0:32, :] >> 16) | (r[32:64, :] & jnp.uint32(0xFFFF0000))
    for c in range(4):
        o_ref[:, 32 * c : 32 * (c + 1)] = w[:, c * _CB : (c + 1) * _CB].T


def _relayout_table(tableT):
    D, V = tableT.shape
    nblk = pl.cdiv(V, 4 * _CB)
    return pl.pallas_call(
        _pack_block,
        grid=(nblk,),
        in_specs=[pl.BlockSpec((D, 4 * _CB), lambda i: (0, i))],
        out_specs=pl.BlockSpec((_CB, 2 * D), lambda i: (i, 0)),
        out_shape=jax.ShapeDtypeStruct((nblk * _CB, 2 * D), jnp.uint32),
    )(tableT)


def _make_pool_kernel(B, L, D, V):
    assert B % NW == 0
    bpw = B // NW
    assert L % 2 == 0
    half = L // 2
    assert half <= 128
    nvec = D // (2 * LANES)

    mesh = plsc.VectorSubcoreMesh(core_axis_name="c", subcore_axis_name="s")

    @functools.partial(
        pl.kernel,
        out_type=jax.ShapeDtypeStruct((B, D), jnp.float32),
        mesh=mesh,
        scratch_types=[
            pltpu.VMEM((bpw, 2, half), jnp.int32),
            pltpu.VMEM((2, 2, half, D // 2), jnp.uint32),
            pltpu.VMEM((bpw, D), jnp.float32),
            pltpu.SemaphoreType.DMA,
            pltpu.SemaphoreType.DMA,
        ],
        compiler_params=pltpu.CompilerParams(use_tc_tiling_on_sc=False),
    )
    def pool(idx_hbm, table_hbm, out_hbm, idx_v, rows_v, acc_v, sem0, sem1):
        wid = lax.axis_index("s") * NC + lax.axis_index("c")
        base = wid * bpw
        pltpu.sync_copy(idx_hbm.at[pl.ds(base, bpw)], idx_v)

        sems = (sem0, sem1)

        def issue(b, buf):
            for c in range(2):
                pltpu.async_copy(
                    table_hbm.at[idx_v.at[b, c]], rows_v.at[buf, c], sems[buf]
                )

        def wait(b, buf):
            for c in range(2):
                pltpu.make_async_copy(
                    table_hbm.at[idx_v.at[b, c]], rows_v.at[buf, c], sems[buf]
                ).wait()

        def reduce_into(buf, b):
            hi_mask = jnp.full((LANES,), 0xFFFF0000, jnp.uint32)

            def body(j, accs):
                out = list(accs)
                for k in range(nvec):
                    for c in range(2):
                        u = rows_v[buf, c, j, pl.ds(LANES * k, LANES)]
                        lo = lax.bitcast_convert_type(u << 16, jnp.float32)
                        hi = lax.bitcast_convert_type(u & hi_mask, jnp.float32)
                        out[k] = out[k] + lo
                        out[nvec + k] = out[nvec + k] + hi
                return tuple(out)

            accs = lax.fori_loop(
                0, half, body,
                tuple(jnp.zeros((LANES,), jnp.float32) for _ in range(2 * nvec)),
            )
            for k in range(2 * nvec):
                acc_v[b, pl.ds(LANES * k, LANES)] = accs[k]

        issue(0, 0)

        def pair(g, carry):
            b0 = 2 * g

            issue(b0 + 1, 1)
            wait(b0, 0)
            reduce_into(0, b0)

            @pl.when(b0 + 2 < bpw)
            def _():
                issue(b0 + 2, 0)

            wait(b0 + 1, 1)
            reduce_into(1, b0 + 1)
            return carry

        lax.fori_loop(0, bpw // 2, pair, 0)

        pltpu.sync_copy(acc_v, out_hbm.at[pl.ds(base, bpw)])

    return pool


def _mlp_block(x_ref, w1_ref, b1_ref, w2_ref, b2_ref, w3_ref, b3_ref, o_ref):
    h = jnp.dot(x_ref[...], w1_ref[...], preferred_element_type=jnp.float32)
    h = jnp.maximum(h + b1_ref[...], 0.0)
    h = jnp.dot(h, w2_ref[...], preferred_element_type=jnp.float32)
    h = jnp.maximum(h + b2_ref[...], 0.0)
    o_ref[...] = (
        jnp.dot(h, w3_ref[...], preferred_element_type=jnp.float32) + b3_ref[...]
    )


def _mlp(pooled, W1, b1, W2, b2, W3, b3):
    B, D = pooled.shape
    H = W1.shape[1]
    O = W3.shape[1]
    BM = 512
    grid = (B // BM,)
    return pl.pallas_call(
        _mlp_block,
        grid=grid,
        in_specs=[
            pl.BlockSpec((BM, D), lambda i: (i, 0)),
            pl.BlockSpec((D, H), lambda i: (0, 0)),
            pl.BlockSpec((1, H), lambda i: (0, 0)),
            pl.BlockSpec((H, H), lambda i: (0, 0)),
            pl.BlockSpec((1, H), lambda i: (0, 0)),
            pl.BlockSpec((H, O), lambda i: (0, 0)),
            pl.BlockSpec((1, O), lambda i: (0, 0)),
        ],
        out_specs=pl.BlockSpec((BM, O), lambda i: (i, 0)),
        out_shape=jax.ShapeDtypeStruct((B, O), jnp.float32),
    )(pooled, W1, b1.reshape(1, H), W2, b2.reshape(1, H), W3, b3.reshape(1, O))


@jax.jit
def kernel(inputs, table, W1, b1, W2, b2, W3, b3):
    B, L = inputs.shape
    V, D = table.shape
    packed = _relayout_table(table.T)
    V4 = 4 * packed.shape[0]
    table_rows = jnp.reshape(packed, (V4, D // 2))
    r = inputs.astype(jnp.int32)
    v = (
        ((r >> (_CBSH + 2)) << (_CBSH + 2))
        + ((r & (_CB - 1)) << 2)
        + ((r >> _CBSH) & 3)
    )
    idx = v.reshape(B, 2, L // 2)
    pooled = _make_pool_kernel(B, L, D, V4)(idx, table_rows)
    return _mlp(pooled, W1, b1, W2, b2, W3, b3)


# speedup vs baseline: 1.9300x; 1.0258x over previous
import functools

import jax
import jax.numpy as jnp
from jax import lax
from jax.experimental import pallas as pl
from jax.experimental.pallas import tpu as pltpu
from jax.experimental.pallas import tpu_sc as plsc

NC = 2
NS = 16
NW = NC * NS
LANES = 16

_CB = 4096
_CBSH = 12


def _pack_block(x_ref, o_ref):
    x = x_ref[...]
    u = lax.bitcast_convert_type(x, jnp.uint32)
    r = u + jnp.uint32(0x7FFF) + ((u >> 16) & jnp.uint32(1))
    w = (r[0:32, :] >> 16) | (r[32:64, :] & jnp.uint32(0xFFFF0000))
    for c in range(4):
        o_ref[:, 32 * c : 32 * (c + 1)] = w[:, c * _CB : (c + 1) * _CB].T


def _relayout_table(tableT):
    D, V = tableT.shape
    nblk = pl.cdiv(V, 4 * _CB)
    return pl.pallas_call(
        _pack_block,
        grid=(nblk,),
        in_specs=[pl.BlockSpec((D, 4 * _CB), lambda i: (0, i))],
        out_specs=pl.BlockSpec((_CB, 2 * D), lambda i: (i, 0)),
        out_shape=jax.ShapeDtypeStruct((nblk * _CB, 2 * D), jnp.uint32),
    )(tableT)


def _make_pool_kernel(B, L, D, V):
    assert B % NW == 0
    bpw = B // NW
    assert L % 2 == 0
    half = L // 2
    assert half <= 128
    nvec = D // (2 * LANES)

    mesh = plsc.VectorSubcoreMesh(core_axis_name="c", subcore_axis_name="s")

    @functools.partial(
        pl.kernel,
        out_type=jax.ShapeDtypeStruct((B, D), jnp.float32),
        mesh=mesh,
        scratch_types=[
            pltpu.VMEM((bpw, 2, half), jnp.int32),
            pltpu.VMEM((2, 2, half, D // 2), jnp.uint32),
            pltpu.VMEM((bpw, D), jnp.float32),
            pltpu.SemaphoreType.DMA,
            pltpu.SemaphoreType.DMA,
        ],
        compiler_params=pltpu.CompilerParams(use_tc_tiling_on_sc=False),
    )
    def pool(idx_hbm, table_hbm, out_hbm, idx_v, rows_v, acc_v, sem0, sem1):
        wid = lax.axis_index("s") * NC + lax.axis_index("c")
        base = wid * bpw
        pltpu.sync_copy(idx_hbm.at[pl.ds(base, bpw)], idx_v)

        sems = (sem0, sem1)

        def issue(b, buf):
            for c in range(2):
                pltpu.async_copy(
                    table_hbm.at[idx_v.at[b, c]], rows_v.at[buf, c], sems[buf]
                )

        def wait(b, buf):
            for c in range(2):
                pltpu.make_async_copy(
                    table_hbm.at[idx_v.at[b, c]], rows_v.at[buf, c], sems[buf]
                ).wait()

        def reduce_into(buf, b):
            hi_mask = jnp.full((LANES,), 0xFFFF0000, jnp.uint32)

            def body(j, accs):
                out = list(accs)
                for k in range(nvec):
                    for c in range(2):
                        u = rows_v[buf, c, j, pl.ds(LANES * k, LANES)]
                        lo = lax.bitcast_convert_type(u << 16, jnp.float32)
                        hi = lax.bitcast_convert_type(u & hi_mask, jnp.float32)
                        out[k] = out[k] + lo
                        out[nvec + k] = out[nvec + k] + hi
                return tuple(out)

            accs = lax.fori_loop(
                0, half, body,
                tuple(jnp.zeros((LANES,), jnp.float32) for _ in range(2 * nvec)),
            )
            for k in range(2 * nvec):
                acc_v[b, pl.ds(LANES * k, LANES)] = accs[k]

        issue(0, 0)

        def pair(g, carry):
            b0 = 2 * g

            issue(b0 + 1, 1)
            wait(b0, 0)
            reduce_into(0, b0)

            @pl.when(b0 + 2 < bpw)
            def _():
                issue(b0 + 2, 0)

            wait(b0 + 1, 1)
            reduce_into(1, b0 + 1)
            return carry

        lax.fori_loop(0, bpw // 2, pair, 0)

        pltpu.sync_copy(acc_v, out_hbm.at[pl.ds(base, bpw)])

    return pool


def _mlp_block(x_ref, w1_ref, b1_ref, w2_ref, b2_ref, w3_ref, b3_ref, o_ref):
    h = jnp.dot(x_ref[...], w1_ref[...], preferred_element_type=jnp.float32)
    h = jnp.maximum(h + b1_ref[...], 0.0)
    h = jnp.dot(h, w2_ref[...], preferred_element_type=jnp.float32)
    h = jnp.maximum(h + b2_ref[...], 0.0)
    o_ref[...] = (
        jnp.dot(h, w3_ref[...], preferred_element_type=jnp.float32) + b3_ref[...]
    )


def _mlp(pooled, W1, b1, W2, b2, W3, b3):
    B, D = pooled.shape
    H = W1.shape[1]
    O = W3.shape[1]
    BM = 512
    grid = (B // BM,)
    return pl.pallas_call(
        _mlp_block,
        grid=grid,
        in_specs=[
            pl.BlockSpec((BM, D), lambda i: (i, 0)),
            pl.BlockSpec((D, H), lambda i: (0, 0)),
            pl.BlockSpec((1, H), lambda i: (0, 0)),
            pl.BlockSpec((H, H), lambda i: (0, 0)),
            pl.BlockSpec((1, H), lambda i: (0, 0)),
            pl.BlockSpec((H, O), lambda i: (0, 0)),
            pl.BlockSpec((1, O), lambda i: (0, 0)),
        ],
        out_specs=pl.BlockSpec((BM, O), lambda i: (i, 0)),
        out_shape=jax.ShapeDtypeStruct((B, O), jnp.float32),
    )(pooled, W1, b1.reshape(1, H), W2, b2.reshape(1, H), W3, b3.reshape(1, O))


@jax.jit
def kernel(inputs, table, W1, b1, W2, b2, W3, b3):
    B, L = inputs.shape
    V, D = table.shape
    packed = _relayout_table(table.T)
    V4 = 4 * packed.shape[0]
    table_rows = jnp.reshape(packed, (V4, D // 2))
    r = inputs.astype(jnp.int32)
    v = (
        ((r >> (_CBSH + 2)) << (_CBSH + 2))
        + ((r & (_CB - 1)) << 2)
        + ((r >> _CBSH) & 3)
    )
    idx = v.reshape(B, 2, L // 2)
    pooled = _make_pool_kernel(B, L, D, V4)(idx, table_rows)
    return _mlp(pooled, W1, b1, W2, b2, W3, b3)


# speedup vs baseline: 1.9424x; 1.0064x over previous
import functools

import jax
import jax.numpy as jnp
from jax import lax
from jax.experimental import pallas as pl
from jax.experimental.pallas import tpu as pltpu
from jax.experimental.pallas import tpu_sc as plsc

NC = 2
NS = 16
NW = NC * NS
LANES = 16

_CB = 8192
_CBSH = 13


def _pack_block(x_ref, o_ref):
    x = x_ref[...]
    u = lax.bitcast_convert_type(x, jnp.uint32)
    r = u + jnp.uint32(0x7FFF) + ((u >> 16) & jnp.uint32(1))
    w = (r[0:32, :] >> 16) | (r[32:64, :] & jnp.uint32(0xFFFF0000))
    for c in range(4):
        o_ref[:, 32 * c : 32 * (c + 1)] = w[:, c * _CB : (c + 1) * _CB].T


def _relayout_table(tableT):
    D, V = tableT.shape
    nblk = pl.cdiv(V, 4 * _CB)
    return pl.pallas_call(
        _pack_block,
        grid=(nblk,),
        in_specs=[pl.BlockSpec((D, 4 * _CB), lambda i: (0, i))],
        out_specs=pl.BlockSpec((_CB, 2 * D), lambda i: (i, 0)),
        out_shape=jax.ShapeDtypeStruct((nblk * _CB, 2 * D), jnp.uint32),
    )(tableT)


def _make_pool_kernel(B, L, D, V):
    assert B % NW == 0
    bpw = B // NW
    assert L % 2 == 0
    half = L // 2
    assert half <= 128
    nvec = D // (2 * LANES)

    mesh = plsc.VectorSubcoreMesh(core_axis_name="c", subcore_axis_name="s")

    @functools.partial(
        pl.kernel,
        out_type=jax.ShapeDtypeStruct((B, D), jnp.float32),
        mesh=mesh,
        scratch_types=[
            pltpu.VMEM((bpw, 2, half), jnp.int32),
            pltpu.VMEM((2, 2, half, D // 2), jnp.uint32),
            pltpu.VMEM((bpw, D), jnp.float32),
            pltpu.SemaphoreType.DMA,
            pltpu.SemaphoreType.DMA,
        ],
        compiler_params=pltpu.CompilerParams(use_tc_tiling_on_sc=False),
    )
    def pool(idx_hbm, table_hbm, out_hbm, idx_v, rows_v, acc_v, sem0, sem1):
        wid = lax.axis_index("s") * NC + lax.axis_index("c")
        base = wid * bpw
        pltpu.sync_copy(idx_hbm.at[pl.ds(base, bpw)], idx_v)

        sems = (sem0, sem1)

        def issue(b, buf):
            for c in range(2):
                pltpu.async_copy(
                    table_hbm.at[idx_v.at[b, c]], rows_v.at[buf, c], sems[buf]
                )

        def wait(b, buf):
            for c in range(2):
                pltpu.make_async_copy(
                    table_hbm.at[idx_v.at[b, c]], rows_v.at[buf, c], sems[buf]
                ).wait()

        def reduce_into(buf, b):
            hi_mask = jnp.full((LANES,), 0xFFFF0000, jnp.uint32)

            def body(j, accs):
                out = list(accs)
                for k in range(nvec):
                    for c in range(2):
                        u = rows_v[buf, c, j, pl.ds(LANES * k, LANES)]
                        lo = lax.bitcast_convert_type(u << 16, jnp.float32)
                        hi = lax.bitcast_convert_type(u & hi_mask, jnp.float32)
                        out[k] = out[k] + lo
                        out[nvec + k] = out[nvec + k] + hi
                return tuple(out)

            accs = lax.fori_loop(
                0, half, body,
                tuple(jnp.zeros((LANES,), jnp.float32) for _ in range(2 * nvec)),
            )
            for k in range(2 * nvec):
                acc_v[b, pl.ds(LANES * k, LANES)] = accs[k]

        issue(0, 0)

        def pair(g, carry):
            b0 = 2 * g

            issue(b0 + 1, 1)
            wait(b0, 0)
            reduce_into(0, b0)

            @pl.when(b0 + 2 < bpw)
            def _():
                issue(b0 + 2, 0)

            wait(b0 + 1, 1)
            reduce_into(1, b0 + 1)
            return carry

        lax.fori_loop(0, bpw // 2, pair, 0)

        pltpu.sync_copy(acc_v, out_hbm.at[pl.ds(base, bpw)])

    return pool


def _mlp_block(x_ref, w1_ref, b1_ref, w2_ref, b2_ref, w3_ref, b3_ref, o_ref):
    h = jnp.dot(x_ref[...], w1_ref[...], preferred_element_type=jnp.float32)
    h = jnp.maximum(h + b1_ref[...], 0.0)
    h = jnp.dot(h, w2_ref[...], preferred_element_type=jnp.float32)
    h = jnp.maximum(h + b2_ref[...], 0.0)
    o_ref[...] = (
        jnp.dot(h, w3_ref[...], preferred_element_type=jnp.float32) + b3_ref[...]
    )


def _mlp(pooled, W1, b1, W2, b2, W3, b3):
    B, D = pooled.shape
    H = W1.shape[1]
    O = W3.shape[1]
    BM = 512
    grid = (B // BM,)
    return pl.pallas_call(
        _mlp_block,
        grid=grid,
        in_specs=[
            pl.BlockSpec((BM, D), lambda i: (i, 0)),
            pl.BlockSpec((D, H), lambda i: (0, 0)),
            pl.BlockSpec((1, H), lambda i: (0, 0)),
            pl.BlockSpec((H, H), lambda i: (0, 0)),
            pl.BlockSpec((1, H), lambda i: (0, 0)),
            pl.BlockSpec((H, O), lambda i: (0, 0)),
            pl.BlockSpec((1, O), lambda i: (0, 0)),
        ],
        out_specs=pl.BlockSpec((BM, O), lambda i: (i, 0)),
        out_shape=jax.ShapeDtypeStruct((B, O), jnp.float32),
    )(pooled, W1, b1.reshape(1, H), W2, b2.reshape(1, H), W3, b3.reshape(1, O))


@jax.jit
def kernel(inputs, table, W1, b1, W2, b2, W3, b3):
    B, L = inputs.shape
    V, D = table.shape
    packed = _relayout_table(table.T)
    V4 = 4 * packed.shape[0]
    table_rows = jnp.reshape(packed, (V4, D // 2))
    r = inputs.astype(jnp.int32)
    v = (
        ((r >> (_CBSH + 2)) << (_CBSH + 2))
        + ((r & (_CB - 1)) << 2)
        + ((r >> _CBSH) & 3)
    )
    idx = v.reshape(B, 2, L // 2)
    pooled = _make_pool_kernel(B, L, D, V4)(idx, table_rows)
    return _mlp(pooled, W1, b1, W2, b2, W3, b3)


# speedup vs baseline: 1.9474x; 1.0026x over previous
import functools

import jax
import jax.numpy as jnp
from jax import lax
from jax.experimental import pallas as pl
from jax.experimental.pallas import tpu as pltpu
from jax.experimental.pallas import tpu_sc as plsc

NC = 2
NS = 16
NW = NC * NS
LANES = 16

_CB = 8192
_CBSH = 13


def _pack_block(x_ref, o_ref):
    x = x_ref[...]
    u = lax.bitcast_convert_type(x, jnp.uint32)
    h = jnp.uint32(0x8000)
    for c in range(4):
        uc = u[:, c * _CB : (c + 1) * _CB]
        w = ((uc[0:32, :] + h) >> 16) | ((uc[32:64, :] + h) & jnp.uint32(0xFFFF0000))
        o_ref[:, 32 * c : 32 * (c + 1)] = w.T


def _relayout_table(tableT):
    D, V = tableT.shape
    nblk = pl.cdiv(V, 4 * _CB)
    return pl.pallas_call(
        _pack_block,
        grid=(nblk,),
        in_specs=[pl.BlockSpec((D, 4 * _CB), lambda i: (0, i))],
        out_specs=pl.BlockSpec((_CB, 2 * D), lambda i: (i, 0)),
        out_shape=jax.ShapeDtypeStruct((nblk * _CB, 2 * D), jnp.uint32),
    )(tableT)


def _make_pool_kernel(B, L, D, V):
    assert B % NW == 0
    bpw = B // NW
    assert L % 2 == 0
    half = L // 2
    assert half <= 128
    nvec = D // (2 * LANES)

    mesh = plsc.VectorSubcoreMesh(core_axis_name="c", subcore_axis_name="s")

    @functools.partial(
        pl.kernel,
        out_type=jax.ShapeDtypeStruct((B, D), jnp.float32),
        mesh=mesh,
        scratch_types=[
            pltpu.VMEM((bpw, 2, half), jnp.int32),
            pltpu.VMEM((2, 2, half, D // 2), jnp.uint32),
            pltpu.VMEM((bpw, D), jnp.float32),
            pltpu.SemaphoreType.DMA,
            pltpu.SemaphoreType.DMA,
        ],
        compiler_params=pltpu.CompilerParams(use_tc_tiling_on_sc=False),
    )
    def pool(idx_hbm, table_hbm, out_hbm, idx_v, rows_v, acc_v, sem0, sem1):
        wid = lax.axis_index("s") * NC + lax.axis_index("c")
        base = wid * bpw
        pltpu.sync_copy(idx_hbm.at[pl.ds(base, bpw)], idx_v)

        sems = (sem0, sem1)

        def issue(b, buf):
            for c in range(2):
                pltpu.async_copy(
                    table_hbm.at[idx_v.at[b, c]], rows_v.at[buf, c], sems[buf]
                )

        def wait(b, buf):
            for c in range(2):
                pltpu.make_async_copy(
                    table_hbm.at[idx_v.at[b, c]], rows_v.at[buf, c], sems[buf]
                ).wait()

        def reduce_into(buf, b):
            hi_mask = jnp.full((LANES,), 0xFFFF0000, jnp.uint32)

            def body(j, accs):
                out = list(accs)
                for k in range(nvec):
                    for c in range(2):
                        u = rows_v[buf, c, j, pl.ds(LANES * k, LANES)]
                        lo = lax.bitcast_convert_type(u << 16, jnp.float32)
                        hi = lax.bitcast_convert_type(u & hi_mask, jnp.float32)
                        out[k] = out[k] + lo
                        out[nvec + k] = out[nvec + k] + hi
                return tuple(out)

            accs = lax.fori_loop(
                0, half, body,
                tuple(jnp.zeros((LANES,), jnp.float32) for _ in range(2 * nvec)),
            )
            for k in range(2 * nvec):
                acc_v[b, pl.ds(LANES * k, LANES)] = accs[k]

        issue(0, 0)

        def pair(g, carry):
            b0 = 2 * g

            issue(b0 + 1, 1)
            wait(b0, 0)
            reduce_into(0, b0)

            @pl.when(b0 + 2 < bpw)
            def _():
                issue(b0 + 2, 0)

            wait(b0 + 1, 1)
            reduce_into(1, b0 + 1)
            return carry

        lax.fori_loop(0, bpw // 2, pair, 0)

        pltpu.sync_copy(acc_v, out_hbm.at[pl.ds(base, bpw)])

    return pool


def _mlp_block(x_ref, w1_ref, b1_ref, w2_ref, b2_ref, w3_ref, b3_ref, o_ref):
    h = jnp.dot(x_ref[...], w1_ref[...], preferred_element_type=jnp.float32)
    h = jnp.maximum(h + b1_ref[...], 0.0)
    h = jnp.dot(h, w2_ref[...], preferred_element_type=jnp.float32)
    h = jnp.maximum(h + b2_ref[...], 0.0)
    o_ref[...] = (
        jnp.dot(h, w3_ref[...], preferred_element_type=jnp.float32) + b3_ref[...]
    )


def _mlp(pooled, W1, b1, W2, b2, W3, b3):
    B, D = pooled.shape
    H = W1.shape[1]
    O = W3.shape[1]
    BM = 512
    grid = (B // BM,)
    return pl.pallas_call(
        _mlp_block,
        grid=grid,
        in_specs=[
            pl.BlockSpec((BM, D), lambda i: (i, 0)),
            pl.BlockSpec((D, H), lambda i: (0, 0)),
            pl.BlockSpec((1, H), lambda i: (0, 0)),
            pl.BlockSpec((H, H), lambda i: (0, 0)),
            pl.BlockSpec((1, H), lambda i: (0, 0)),
            pl.BlockSpec((H, O), lambda i: (0, 0)),
            pl.BlockSpec((1, O), lambda i: (0, 0)),
        ],
        out_specs=pl.BlockSpec((BM, O), lambda i: (i, 0)),
        out_shape=jax.ShapeDtypeStruct((B, O), jnp.float32),
    )(pooled, W1, b1.reshape(1, H), W2, b2.reshape(1, H), W3, b3.reshape(1, O))


@jax.jit
def kernel(inputs, table, W1, b1, W2, b2, W3, b3):
    B, L = inputs.shape
    V, D = table.shape
    packed = _relayout_table(table.T)
    V4 = 4 * packed.shape[0]
    table_rows = jnp.reshape(packed, (V4, D // 2))
    r = inputs.astype(jnp.int32)
    v = (
        ((r >> (_CBSH + 2)) << (_CBSH + 2))
        + ((r & (_CB - 1)) << 2)
        + ((r >> _CBSH) & 3)
    )
    idx = v.reshape(B, 2, L // 2)
    pooled = _make_pool_kernel(B, L, D, V4)(idx, table_rows)
    return _mlp(pooled, W1, b1, W2, b2, W3, b3)


# speedup vs baseline: 2.1280x; 1.0928x over previous
import functools

import jax
import jax.numpy as jnp
from jax import lax
from jax.experimental import pallas as pl
from jax.experimental.pallas import tpu as pltpu
from jax.experimental.pallas import tpu_sc as plsc

NC = 2
NS = 16
NW = NC * NS
LANES = 16

_CB = 8192
_CBSH = 13


def _pack_block(x_ref, o_ref):
    x = x_ref[...]
    u = lax.bitcast_convert_type(x, jnp.uint32)
    h = jnp.uint32(0x8000)
    for c in range(4):
        uc = u[:, c * _CB : (c + 1) * _CB]
        w = ((uc[0:32, :] + h) >> 16) | ((uc[32:64, :] + h) & jnp.uint32(0xFFFF0000))
        o_ref[:, 32 * c : 32 * (c + 1)] = w.T


def _relayout_table(tableT):
    D, V = tableT.shape
    nblk = pl.cdiv(V, 4 * _CB)
    return pl.pallas_call(
        _pack_block,
        grid=(nblk,),
        in_specs=[pl.BlockSpec((D, 4 * _CB), lambda i: (0, i))],
        out_specs=pl.BlockSpec((_CB, 2 * D), lambda i: (i, 0)),
        out_shape=jax.ShapeDtypeStruct((nblk * _CB, 2 * D), jnp.uint32),
        compiler_params=pltpu.CompilerParams(
            dimension_semantics=("parallel",)
        ),
    )(tableT)


def _make_pool_kernel(B, L, D, V):
    assert B % NW == 0
    bpw = B // NW
    assert L % 2 == 0
    half = L // 2
    assert half <= 128
    nvec = D // (2 * LANES)

    mesh = plsc.VectorSubcoreMesh(core_axis_name="c", subcore_axis_name="s")

    @functools.partial(
        pl.kernel,
        out_type=jax.ShapeDtypeStruct((B, D), jnp.float32),
        mesh=mesh,
        scratch_types=[
            pltpu.VMEM((bpw, 2, half), jnp.int32),
            pltpu.VMEM((4, 2, half, D // 2), jnp.uint32),
            pltpu.VMEM((bpw, D), jnp.float32),
            pltpu.SemaphoreType.DMA,
            pltpu.SemaphoreType.DMA,
            pltpu.SemaphoreType.DMA,
            pltpu.SemaphoreType.DMA,
        ],
        compiler_params=pltpu.CompilerParams(use_tc_tiling_on_sc=False),
    )
    def pool(
        idx_hbm, table_hbm, out_hbm, idx_v, rows_v, acc_v,
        sem0, sem1, sem2, sem3,
    ):
        wid = lax.axis_index("s") * NC + lax.axis_index("c")
        base = wid * bpw
        pltpu.sync_copy(idx_hbm.at[pl.ds(base, bpw)], idx_v)

        sems = (sem0, sem1, sem2, sem3)

        def issue(b, buf):
            for c in range(2):
                pltpu.async_copy(
                    table_hbm.at[idx_v.at[b, c]], rows_v.at[buf, c], sems[buf]
                )

        def wait(b, buf):
            for c in range(2):
                pltpu.make_async_copy(
                    table_hbm.at[idx_v.at[b, c]], rows_v.at[buf, c], sems[buf]
                ).wait()

        def reduce_into(buf, b):
            hi_mask = jnp.full((LANES,), 0xFFFF0000, jnp.uint32)

            def body(j, accs):
                out = list(accs)
                for k in range(nvec):
                    for c in range(2):
                        u = rows_v[buf, c, j, pl.ds(LANES * k, LANES)]
                        lo = lax.bitcast_convert_type(u << 16, jnp.float32)
                        hi = lax.bitcast_convert_type(u & hi_mask, jnp.float32)
                        out[k] = out[k] + lo
                        out[nvec + k] = out[nvec + k] + hi
                return tuple(out)

            accs = lax.fori_loop(
                0, half, body,
                tuple(jnp.zeros((LANES,), jnp.float32) for _ in range(2 * nvec)),
            )
            for k in range(2 * nvec):
                acc_v[b, pl.ds(LANES * k, LANES)] = accs[k]

        issue(0, 0)
        issue(1, 1)
        issue(2, 2)

        def quad(g, carry):
            b0 = 4 * g
            for ph in range(4):
                b = b0 + ph
                wait(b, ph)
                nxt = b + 3

                @pl.when(nxt < bpw)
                def _(nxt=nxt, nbuf=(ph + 3) % 4):
                    issue(nxt, nbuf)

                reduce_into(ph, b)
            return carry

        lax.fori_loop(0, bpw // 4, quad, 0)

        pltpu.sync_copy(acc_v, out_hbm.at[pl.ds(base, bpw)])

    return pool


def _mlp_block(x_ref, w1_ref, b1_ref, w2_ref, b2_ref, w3_ref, b3_ref, o_ref):
    h = jnp.dot(x_ref[...], w1_ref[...], preferred_element_type=jnp.float32)
    h = jnp.maximum(h + b1_ref[...], 0.0)
    h = jnp.dot(h, w2_ref[...], preferred_element_type=jnp.float32)
    h = jnp.maximum(h + b2_ref[...], 0.0)
    o_ref[...] = (
        jnp.dot(h, w3_ref[...], preferred_element_type=jnp.float32) + b3_ref[...]
    )


def _mlp(pooled, W1, b1, W2, b2, W3, b3):
    B, D = pooled.shape
    H = W1.shape[1]
    O = W3.shape[1]
    BM = 512
    grid = (B // BM,)
    return pl.pallas_call(
        _mlp_block,
        grid=grid,
        in_specs=[
            pl.BlockSpec((BM, D), lambda i: (i, 0)),
            pl.BlockSpec((D, H), lambda i: (0, 0)),
            pl.BlockSpec((1, H), lambda i: (0, 0)),
            pl.BlockSpec((H, H), lambda i: (0, 0)),
            pl.BlockSpec((1, H), lambda i: (0, 0)),
            pl.BlockSpec((H, O), lambda i: (0, 0)),
            pl.BlockSpec((1, O), lambda i: (0, 0)),
        ],
        out_specs=pl.BlockSpec((BM, O), lambda i: (i, 0)),
        out_shape=jax.ShapeDtypeStruct((B, O), jnp.float32),
    )(pooled, W1, b1.reshape(1, H), W2, b2.reshape(1, H), W3, b3.reshape(1, O))


@jax.jit
def kernel(inputs, table, W1, b1, W2, b2, W3, b3):
    B, L = inputs.shape
    V, D = table.shape
    packed = _relayout_table(table.T)
    V4 = 4 * packed.shape[0]
    table_rows = jnp.reshape(packed, (V4, D // 2))
    r = inputs.astype(jnp.int32)
    v = (
        ((r >> (_CBSH + 2)) << (_CBSH + 2))
        + ((r & (_CB - 1)) << 2)
        + ((r >> _CBSH) & 3)
    )
    idx = v.reshape(B, 2, L // 2)
    pooled = _make_pool_kernel(B, L, D, V4)(idx, table_rows)
    return _mlp(pooled, W1, b1, W2, b2, W3, b3)


# speedup vs baseline: 2.1330x; 1.0023x over previous
import functools

import jax
import jax.numpy as jnp
from jax import lax
from jax.experimental import pallas as pl
from jax.experimental.pallas import tpu as pltpu
from jax.experimental.pallas import tpu_sc as plsc

NC = 2
NS = 16
NW = NC * NS
LANES = 16

_CB = 8192
_CBSH = 13


def _pack_block(x_ref, o_ref):
    x = x_ref[...]
    u = lax.bitcast_convert_type(x, jnp.uint32)
    h = jnp.uint32(0x8000)
    for c in range(4):
        uc = u[:, c * _CB : (c + 1) * _CB]
        w = ((uc[0:32, :] + h) >> 16) | ((uc[32:64, :] + h) & jnp.uint32(0xFFFF0000))
        o_ref[:, 32 * c : 32 * (c + 1)] = w.T


def _relayout_table(tableT):
    D, V = tableT.shape
    nblk = pl.cdiv(V, 4 * _CB)
    return pl.pallas_call(
        _pack_block,
        grid=(nblk,),
        in_specs=[pl.BlockSpec((D, 4 * _CB), lambda i: (0, i))],
        out_specs=pl.BlockSpec((_CB, 2 * D), lambda i: (i, 0)),
        out_shape=jax.ShapeDtypeStruct((nblk * _CB, 2 * D), jnp.uint32),
        compiler_params=pltpu.CompilerParams(
            dimension_semantics=("parallel",)
        ),
    )(tableT)


def _make_pool_kernel(B, L, D, V):
    assert B % NW == 0
    bpw = B // NW
    assert L % 2 == 0
    half = L // 2
    assert half <= 128
    nvec = D // (2 * LANES)

    mesh = plsc.VectorSubcoreMesh(core_axis_name="c", subcore_axis_name="s")

    @functools.partial(
        pl.kernel,
        out_type=jax.ShapeDtypeStruct((B, D), jnp.float32),
        mesh=mesh,
        scratch_types=[
            pltpu.VMEM((bpw, 2, half), jnp.int32),
            pltpu.VMEM((4, 2, half, D // 2), jnp.uint32),
            pltpu.VMEM((bpw, D), jnp.float32),
            pltpu.SemaphoreType.DMA,
            pltpu.SemaphoreType.DMA,
            pltpu.SemaphoreType.DMA,
            pltpu.SemaphoreType.DMA,
        ],
        compiler_params=pltpu.CompilerParams(use_tc_tiling_on_sc=False),
    )
    def pool(
        idx_hbm, table_hbm, out_hbm, idx_v, rows_v, acc_v,
        sem0, sem1, sem2, sem3,
    ):
        wid = lax.axis_index("s") * NC + lax.axis_index("c")
        base = wid * bpw
        pltpu.sync_copy(idx_hbm.at[pl.ds(base, bpw)], idx_v)

        sems = (sem0, sem1, sem2, sem3)

        def issue(b, buf):
            for c in range(2):
                pltpu.async_copy(
                    table_hbm.at[idx_v.at[b, c]], rows_v.at[buf, c], sems[buf]
                )

        def wait(b, buf):
            for c in range(2):
                pltpu.make_async_copy(
                    table_hbm.at[idx_v.at[b, c]], rows_v.at[buf, c], sems[buf]
                ).wait()

        def reduce_into(buf, b):
            hi_mask = jnp.full((LANES,), 0xFFFF0000, jnp.uint32)

            def body(j, accs):
                out = list(accs)
                for k in range(nvec):
                    for c in range(2):
                        u = rows_v[buf, c, j, pl.ds(LANES * k, LANES)]
                        lo = lax.bitcast_convert_type(u << 16, jnp.float32)
                        hi = lax.bitcast_convert_type(u & hi_mask, jnp.float32)
                        out[k] = out[k] + lo
                        out[nvec + k] = out[nvec + k] + hi
                return tuple(out)

            accs = lax.fori_loop(
                0, half, body,
                tuple(jnp.zeros((LANES,), jnp.float32) for _ in range(2 * nvec)),
            )
            for k in range(2 * nvec):
                acc_v[b, pl.ds(LANES * k, LANES)] = accs[k]

        issue(0, 0)
        issue(1, 1)
        issue(2, 2)

        def quad(g, carry):
            b0 = 4 * g
            for ph in range(4):
                b = b0 + ph
                wait(b, ph)
                nxt = b + 3

                @pl.when(nxt < bpw)
                def _(nxt=nxt, nbuf=(ph + 3) % 4):
                    issue(nxt, nbuf)

                reduce_into(ph, b)
            return carry

        lax.fori_loop(0, bpw // 4, quad, 0)

        pltpu.sync_copy(acc_v, out_hbm.at[pl.ds(base, bpw)])

    return pool


def _mlp_block(x_ref, w1_ref, b1_ref, w2_ref, b2_ref, w3_ref, b3_ref, o_ref):
    h = jnp.dot(x_ref[...], w1_ref[...], preferred_element_type=jnp.float32)
    h = jnp.maximum(h + b1_ref[...], 0.0)
    h = jnp.dot(h, w2_ref[...], preferred_element_type=jnp.float32)
    h = jnp.maximum(h + b2_ref[...], 0.0)
    o_ref[...] = (
        jnp.dot(h, w3_ref[...], preferred_element_type=jnp.float32) + b3_ref[...]
    )


def _mlp(pooled, W1, b1, W2, b2, W3, b3):
    B, D = pooled.shape
    H = W1.shape[1]
    O = W3.shape[1]
    BM = 512
    grid = (B // BM,)
    return pl.pallas_call(
        _mlp_block,
        grid=grid,
        in_specs=[
            pl.BlockSpec((BM, D), lambda i: (i, 0)),
            pl.BlockSpec((D, H), lambda i: (0, 0)),
            pl.BlockSpec((1, H), lambda i: (0, 0)),
            pl.BlockSpec((H, H), lambda i: (0, 0)),
            pl.BlockSpec((1, H), lambda i: (0, 0)),
            pl.BlockSpec((H, O), lambda i: (0, 0)),
            pl.BlockSpec((1, O), lambda i: (0, 0)),
        ],
        out_specs=pl.BlockSpec((BM, O), lambda i: (i, 0)),
        out_shape=jax.ShapeDtypeStruct((B, O), jnp.float32),
        compiler_params=pltpu.CompilerParams(
            dimension_semantics=("parallel",)
        ),
    )(pooled, W1, b1.reshape(1, H), W2, b2.reshape(1, H), W3, b3.reshape(1, O))


@jax.jit
def kernel(inputs, table, W1, b1, W2, b2, W3, b3):
    B, L = inputs.shape
    V, D = table.shape
    packed = _relayout_table(table.T)
    V4 = 4 * packed.shape[0]
    table_rows = jnp.reshape(packed, (V4, D // 2))
    r = inputs.astype(jnp.int32)
    v = (
        ((r >> (_CBSH + 2)) << (_CBSH + 2))
        + ((r & (_CB - 1)) << 2)
        + ((r >> _CBSH) & 3)
    )
    idx = v.reshape(B, 2, L // 2)
    pooled = _make_pool_kernel(B, L, D, V4)(idx, table_rows)
    return _mlp(pooled, W1, b1, W2, b2, W3, b3)
